# Initial kernel scaffold; baseline (speedup 1.0000x reference)
import jax
import jax.numpy as jnp
from jax import lax
from jax.experimental import pallas as pl
from jax.experimental.pallas import tpu as pltpu
from jax.experimental.pallas import tpu_sc as plsc

N_USER = 5000
N_NODES = 10000
N_PAD = 10240
N_EDGES = 320000
D = 64
DEGW = 16

NC, NS = 2, 16
NW = NC * NS
EPW = N_EDGES // NW
B = 80
CH = EPW // B
K = 5
ROWS_PT = N_PAD // NS

_f32 = jnp.float32


def _sc_scatter_body(g_hbm, src_hbm, dst_hbm, out_hbm,
                     ssrc, sdst, rows, tbuf, acc, gsem, ssem):
    c = lax.axis_index("c")
    s = lax.axis_index("s")
    wid = c * NS + s

    pltpu.sync_copy(src_hbm.at[pl.ds(wid * CH, CH)], ssrc)
    pltpu.sync_copy(dst_hbm.at[pl.ds(wid * CH, CH)], sdst)

    z = jnp.zeros((16,), _f32)
    zref = rows.at[0]

    def _zrow(i, carry):
        for j in range(D // 16):
            zref[i, pl.ds(j * 16, 16)] = z
        return carry

    lax.fori_loop(0, B, _zrow, 0)
    r0 = s * ROWS_PT
    for j in range(ROWS_PT // B):
        pltpu.sync_copy(rows.at[0], acc.at[pl.ds(r0 + j * B, B)])
    plsc.subcore_barrier()

    def _step(t, carry):
        gd = []
        for k in range(K):
            ch = t * K + k
            gd.append(pltpu.async_copy(g_hbm.at[ssrc.at[ch]], rows.at[k], gsem))
        for dd in gd:
            dd.wait()
        sd = []
        for k in range(K):
            ch = t * K + k
            sd.append(pltpu.async_copy(rows.at[k], acc.at[sdst.at[ch]],
                                       ssem, add=True))
        for dd in sd:
            dd.wait()
        return carry

    lax.fori_loop(0, CH // K, _step, 0)
    plsc.subcore_barrier()

    pltpu.sync_copy(acc.at[pl.ds(r0, ROWS_PT)], tbuf)
    pltpu.sync_copy(tbuf, out_hbm.at[c, pl.ds(r0, ROWS_PT)])


def _sc_degree_body(dst_hbm, out_hbm, sdst, ones, dbuf, dacc, sem):
    c = lax.axis_index("c")
    s = lax.axis_index("s")
    wid = c * NS + s

    pltpu.sync_copy(dst_hbm.at[pl.ds(wid * CH, CH)], sdst)

    one16 = jnp.full((16,), 1.0, _f32)
    z16 = jnp.zeros((16,), _f32)

    def _fill(i, carry):
        ones[i, pl.ds(0, 16)] = one16
        return carry

    lax.fori_loop(0, B, _fill, 0)

    def _zero(i, carry):
        dbuf[i, pl.ds(0, 16)] = z16
        return carry

    lax.fori_loop(0, ROWS_PT, _zero, 0)
    r0 = s * ROWS_PT
    pltpu.sync_copy(dbuf, dacc.at[pl.ds(r0, ROWS_PT)])
    plsc.subcore_barrier()

    def _step(t, carry):
        sd = []
        for k in range(K):
            ch = t * K + k
            sd.append(pltpu.async_copy(ones, dacc.at[sdst.at[ch]],
                                       sem, add=True))
        for dd in sd:
            dd.wait()
        return carry

    lax.fori_loop(0, CH // K, _step, 0)
    plsc.subcore_barrier()

    pltpu.sync_copy(dacc.at[pl.ds(r0, ROWS_PT)], dbuf)
    pltpu.sync_copy(dbuf, out_hbm.at[c, pl.ds(r0, ROWS_PT)])


def _tc_embed_body(feat_ref, wm_ref, bm_ref, pref_ref, x_ref):
    nf = jnp.dot(feat_ref[...], wm_ref[...],
                 preferred_element_type=_f32) + bm_ref[...]
    x = jnp.concatenate([pref_ref[...], nf], axis=0)
    nrm = jnp.maximum(jnp.sqrt(jnp.sum(x * x, axis=1, keepdims=True)), 1e-12)
    x_ref[...] = x / nrm


def _tc_g1_body(x_ref, w1_ref, degt_ref, g1_ref, dis_ref):
    deg = degt_ref[0:N_NODES, 0:1] + degt_ref[0:N_NODES, 1:2]
    dis = jnp.where(deg > 0.0, lax.rsqrt(jnp.maximum(deg, 1e-12)), 0.0)
    g1_ref[...] = dis * jnp.dot(x_ref[...], w1_ref[...],
                                preferred_element_type=_f32)
    dis_ref[...] = dis


def _tc_g2_body(s1_ref, dis_ref, w2_ref, p1_ref, p2_ref, g2_ref):
    dis = dis_ref[...]
    t = dis * (s1_ref[0, 0:N_NODES, :] + s1_ref[1, 0:N_NODES, :])
    scale = p1_ref[0, 0] * p2_ref[0, 0]
    g2_ref[...] = (scale * dis) * jnp.dot(t, w2_ref[...],
                                          preferred_element_type=_f32)


def _tc_final_body(s2_ref, dis_ref, x2_ref):
    x2_ref[...] = dis_ref[...] * (s2_ref[0, 0:N_NODES, :]
                                  + s2_ref[1, 0:N_NODES, :])


def _sc_mesh():
    return plsc.VectorSubcoreMesh(core_axis_name="c", subcore_axis_name="s")


def _sc_scatter(g, src, dst):
    return pl.kernel(
        _sc_scatter_body,
        out_type=jax.ShapeDtypeStruct((NC, N_PAD, D), _f32),
        mesh=_sc_mesh(),
        scratch_types=[
            pltpu.VMEM((CH, B), jnp.int32),
            pltpu.VMEM((CH, B), jnp.int32),
            pltpu.VMEM((K, B, D), _f32),
            pltpu.VMEM((ROWS_PT, D), _f32),
            pltpu.VMEM_SHARED((N_PAD, D), _f32),
            pltpu.SemaphoreType.DMA,
            pltpu.SemaphoreType.DMA,
        ],
    )(g, src, dst)


def _sc_degree(dst):
    return pl.kernel(
        _sc_degree_body,
        out_type=jax.ShapeDtypeStruct((NC, N_PAD, DEGW), _f32),
        mesh=_sc_mesh(),
        scratch_types=[
            pltpu.VMEM((CH, B), jnp.int32),
            pltpu.VMEM((B, DEGW), _f32),
            pltpu.VMEM((ROWS_PT, DEGW), _f32),
            pltpu.VMEM_SHARED((N_PAD, DEGW), _f32),
            pltpu.SemaphoreType.DMA,
        ],
    )(dst)


def kernel(features, edge_index, preference, W_mlp, b_mlp, W1, p1, W2, p2):
    src = edge_index[0].reshape(NW * CH, B)
    dst = edge_index[1].reshape(NW * CH, B)

    deg_parts = _sc_degree(dst)
    x = pl.pallas_call(
        _tc_embed_body,
        out_shape=jax.ShapeDtypeStruct((N_NODES, D), _f32),
    )(features, W_mlp, b_mlp.reshape(1, D), preference)

    degt = deg_parts[:, :, 0].T
    g1, dis = pl.pallas_call(
        _tc_g1_body,
        out_shape=[jax.ShapeDtypeStruct((N_NODES, D), _f32),
                   jax.ShapeDtypeStruct((N_NODES, 1), _f32)],
    )(x, W1, degt)

    s1 = _sc_scatter(g1, src, dst)
    g2 = pl.pallas_call(
        _tc_g2_body,
        out_shape=jax.ShapeDtypeStruct((N_NODES, D), _f32),
    )(s1, dis, W2, p1.reshape(1, 1), p2.reshape(1, 1))

    s2 = _sc_scatter(g2, src, dst)
    x2 = pl.pallas_call(
        _tc_final_body,
        out_shape=jax.ShapeDtypeStruct((N_NODES, D), _f32),
    )(s2, dis)
    return (x2, p2)


# speedup vs baseline: 23.3028x; 23.3028x over previous
import jax
import jax.numpy as jnp
from jax import lax
from jax.experimental import pallas as pl
from jax.experimental.pallas import tpu as pltpu
from jax.experimental.pallas import tpu_sc as plsc

N_USER = 5000
N_NODES = 10000
N_PAD = 10240
N_EDGES = 320000
D = 64
DEGW = 16

NC, NS = 2, 16
NW = NC * NS
EPW = N_EDGES // NW
B = 80
CH = EPW // B
K = 5
ROWS_PT = N_PAD // NS

_f32 = jnp.float32


def _sc_scatter_body(g_hbm, src_hbm, dst_hbm, out_hbm,
                     ssrc, sdst, rows, tbuf, acc, gsem, ssem):
    c = lax.axis_index("c")
    s = lax.axis_index("s")
    wid = c * NS + s

    pltpu.sync_copy(src_hbm.at[wid], ssrc)
    pltpu.sync_copy(dst_hbm.at[wid], sdst)

    z = jnp.zeros((16,), _f32)
    zref = rows.at[0]

    def _zrow(i, carry):
        for j in range(D // 16):
            zref[i, pl.ds(j * 16, 16)] = z
        return carry

    lax.fori_loop(0, B, _zrow, 0)
    r0 = s * ROWS_PT
    for j in range(ROWS_PT // B):
        pltpu.sync_copy(rows.at[0], acc.at[pl.ds(r0 + j * B, B)])
    plsc.subcore_barrier()

    def _step(t, carry):
        gd = []
        for k in range(K):
            ch = t * K + k
            gd.append(pltpu.async_copy(g_hbm.at[ssrc.at[ch]], rows.at[k], gsem))
        for dd in gd:
            dd.wait()
        sd = []
        for k in range(K):
            ch = t * K + k
            sd.append(pltpu.async_copy(rows.at[k], acc.at[sdst.at[ch]],
                                       ssem, add=True))
        for dd in sd:
            dd.wait()
        return carry

    lax.fori_loop(0, CH // K, _step, 0)
    plsc.subcore_barrier()

    pltpu.sync_copy(acc.at[pl.ds(r0, ROWS_PT)], tbuf)
    pltpu.sync_copy(tbuf, out_hbm.at[c, pl.ds(r0, ROWS_PT)])


def _sc_degree_body(dst_hbm, out_hbm, sdst, ones, dbuf, dacc, sem):
    c = lax.axis_index("c")
    s = lax.axis_index("s")
    wid = c * NS + s

    pltpu.sync_copy(dst_hbm.at[wid], sdst)

    one16 = jnp.full((16,), 1.0, _f32)
    z16 = jnp.zeros((16,), _f32)

    def _fill(i, carry):
        ones[i, pl.ds(0, 16)] = one16
        return carry

    lax.fori_loop(0, B, _fill, 0)

    def _zero(i, carry):
        dbuf[i, pl.ds(0, 16)] = z16
        return carry

    lax.fori_loop(0, ROWS_PT, _zero, 0)
    r0 = s * ROWS_PT
    pltpu.sync_copy(dbuf, dacc.at[pl.ds(r0, ROWS_PT)])
    plsc.subcore_barrier()

    def _step(t, carry):
        sd = []
        for k in range(K):
            ch = t * K + k
            sd.append(pltpu.async_copy(ones, dacc.at[sdst.at[ch]],
                                       sem, add=True))
        for dd in sd:
            dd.wait()
        return carry

    lax.fori_loop(0, CH // K, _step, 0)
    plsc.subcore_barrier()

    pltpu.sync_copy(dacc.at[pl.ds(r0, ROWS_PT)], dbuf)
    pltpu.sync_copy(dbuf, out_hbm.at[c, pl.ds(r0, ROWS_PT)])


def _tc_embed_body(feat_ref, wm_ref, bm_ref, pref_ref, x_ref):
    nf = jnp.dot(feat_ref[...], wm_ref[...],
                 preferred_element_type=_f32) + bm_ref[...]
    x = jnp.concatenate([pref_ref[...], nf], axis=0)
    nrm = jnp.maximum(jnp.sqrt(jnp.sum(x * x, axis=1, keepdims=True)), 1e-12)
    x_ref[...] = x / nrm


def _tc_g1_body(x_ref, w1_ref, degt_ref, g1_ref, dis_ref):
    deg = degt_ref[0:N_NODES, 0:1] + degt_ref[0:N_NODES, 1:2]
    dis = jnp.where(deg > 0.0, lax.rsqrt(jnp.maximum(deg, 1e-12)), 0.0)
    g1_ref[...] = dis * jnp.dot(x_ref[...], w1_ref[...],
                                preferred_element_type=_f32)
    dis_ref[...] = dis


def _tc_g2_body(s1_ref, dis_ref, w2_ref, p1_ref, p2_ref, g2_ref):
    dis = dis_ref[...]
    t = dis * (s1_ref[0, 0:N_NODES, :] + s1_ref[1, 0:N_NODES, :])
    scale = p1_ref[0, 0] * p2_ref[0, 0]
    g2_ref[...] = (scale * dis) * jnp.dot(t, w2_ref[...],
                                          preferred_element_type=_f32)


def _tc_final_body(s2_ref, dis_ref, x2_ref):
    x2_ref[...] = dis_ref[...] * (s2_ref[0, 0:N_NODES, :]
                                  + s2_ref[1, 0:N_NODES, :])


def _sc_mesh():
    return plsc.VectorSubcoreMesh(core_axis_name="c", subcore_axis_name="s")


def _sc_scatter(g, src, dst):
    return pl.kernel(
        _sc_scatter_body,
        out_type=jax.ShapeDtypeStruct((NC, N_PAD, D), _f32),
        mesh=_sc_mesh(),
        scratch_types=[
            pltpu.VMEM((CH, B), jnp.int32),
            pltpu.VMEM((CH, B), jnp.int32),
            pltpu.VMEM((K, B, D), _f32),
            pltpu.VMEM((ROWS_PT, D), _f32),
            pltpu.VMEM_SHARED((N_PAD, D), _f32),
            pltpu.SemaphoreType.DMA,
            pltpu.SemaphoreType.DMA,
        ],
        compiler_params=pltpu.CompilerParams(use_tc_tiling_on_sc=False),
    )(g, src, dst)


def _sc_degree(dst):
    return pl.kernel(
        _sc_degree_body,
        out_type=jax.ShapeDtypeStruct((NC, N_PAD, DEGW), _f32),
        mesh=_sc_mesh(),
        scratch_types=[
            pltpu.VMEM((CH, B), jnp.int32),
            pltpu.VMEM((B, DEGW), _f32),
            pltpu.VMEM((ROWS_PT, DEGW), _f32),
            pltpu.VMEM_SHARED((N_PAD, DEGW), _f32),
            pltpu.SemaphoreType.DMA,
        ],
        compiler_params=pltpu.CompilerParams(use_tc_tiling_on_sc=False),
    )(dst)


def kernel(features, edge_index, preference, W_mlp, b_mlp, W1, p1, W2, p2):
    src = edge_index[0].reshape(NW, CH, B)
    dst = edge_index[1].reshape(NW, CH, B)

    deg_parts = _sc_degree(dst)
    x = pl.pallas_call(
        _tc_embed_body,
        out_shape=jax.ShapeDtypeStruct((N_NODES, D), _f32),
    )(features, W_mlp, b_mlp.reshape(1, D), preference)

    degt = deg_parts[:, :, 0].T
    g1, dis = pl.pallas_call(
        _tc_g1_body,
        out_shape=[jax.ShapeDtypeStruct((N_NODES, D), _f32),
                   jax.ShapeDtypeStruct((N_NODES, 1), _f32)],
    )(x, W1, degt)

    s1 = _sc_scatter(g1, src, dst)
    g2 = pl.pallas_call(
        _tc_g2_body,
        out_shape=jax.ShapeDtypeStruct((N_NODES, D), _f32),
    )(s1, dis, W2, p1.reshape(1, 1), p2.reshape(1, 1))

    s2 = _sc_scatter(g2, src, dst)
    x2 = pl.pallas_call(
        _tc_final_body,
        out_shape=jax.ShapeDtypeStruct((N_NODES, D), _f32),
    )(s2, dis)
    return (x2, p2)


# speedup vs baseline: 40.9934x; 1.7592x over previous
import jax
import jax.numpy as jnp
from jax import lax
from jax.experimental import pallas as pl
from jax.experimental.pallas import tpu as pltpu
from jax.experimental.pallas import tpu_sc as plsc

N_USER = 5000
N_NODES = 10000
N_PAD = 10240
N_EDGES = 320000
D = 64
DEGW = 16

NC, NS = 2, 16
NW = NC * NS
B = 128
E_PAD = 327680
PAD_NODE = 10000
EPW = E_PAD // NW
CH = EPW // B
K = 4
ROWS_PT = N_PAD // NS

_f32 = jnp.float32


def _sc_scatter_body(g_hbm, src_hbm, dst_hbm, out_hbm,
                     ssrc, sdst, rows, acc, gsem, ssem):
    c = lax.axis_index("c")
    s = lax.axis_index("s")
    wid = c * NS + s

    pltpu.sync_copy(src_hbm.at[pl.ds(wid * CH, CH)], ssrc)
    pltpu.sync_copy(dst_hbm.at[pl.ds(wid * CH, CH)], sdst)

    z = jnp.zeros((16,), _f32)
    zref = rows.at[0, 0]

    def _zrow(i, carry):
        for j in range(D // 16):
            zref[i, pl.ds(j * 16, 16)] = z
        return carry

    lax.fori_loop(0, B, _zrow, 0)
    r0 = s * ROWS_PT
    zz = []
    for j in range(ROWS_PT // B):
        zz.append(pltpu.async_copy(rows.at[0, 0],
                                   acc.at[pl.ds(r0 + j * B, B)], gsem))
    for dd in zz:
        dd.wait()
    plsc.subcore_barrier()

    GR = CH // K

    def _drain_gathers(bank):
        for k in range(K):
            pltpu.make_async_copy(g_hbm.at[pl.ds(0, B)],
                                  rows.at[bank, k], gsem).wait()

    def _drain_scatters(bank):
        for k in range(K):
            pltpu.make_async_copy(rows.at[bank, k],
                                  acc.at[pl.ds(0, B)], ssem).wait()

    def _issue_gathers(t, bank):
        for k in range(K):
            pltpu.async_copy(g_hbm.at[ssrc.at[t * K + k]],
                             rows.at[bank, k], gsem)

    def _issue_scatters(t, bank):
        for k in range(K):
            pltpu.async_copy(rows.at[bank, k], acc.at[sdst.at[t * K + k]],
                             ssem, add=True)

    _issue_gathers(0, 0)

    def _step(t, carry):
        b = lax.rem(t, 2)
        b2 = 1 - b

        @pl.when(t >= 1)
        def _():
            _drain_scatters(b2)

        @pl.when(t <= GR - 2)
        def _():
            _issue_gathers(t + 1, b2)

        _drain_gathers(b)
        _issue_scatters(t, b)
        return carry

    lax.fori_loop(0, GR, _step, 0)
    _drain_scatters(lax.rem(GR - 1, 2))
    plsc.subcore_barrier()

    rd = []
    for j in range(ROWS_PT // B):
        rd.append(pltpu.async_copy(acc.at[pl.ds(r0 + j * B, B)],
                                   rows.at[j // K, j % K], gsem))
    for dd in rd:
        dd.wait()
    wr = []
    for j in range(ROWS_PT // B):
        wr.append(pltpu.async_copy(rows.at[j // K, j % K],
                                   out_hbm.at[c, pl.ds(r0 + j * B, B)], ssem))
    for dd in wr:
        dd.wait()


def _sc_degree_body(dst_hbm, out_hbm, sdst, ones, dbuf, dacc, sem):
    c = lax.axis_index("c")
    s = lax.axis_index("s")
    wid = c * NS + s

    pltpu.sync_copy(dst_hbm.at[pl.ds(wid * CH, CH)], sdst)

    one16 = jnp.full((16,), 1.0, _f32)
    z16 = jnp.zeros((16,), _f32)

    def _fill(i, carry):
        ones[i, pl.ds(0, 16)] = one16
        return carry

    lax.fori_loop(0, B, _fill, 0)

    def _zero(i, carry):
        dbuf[i, pl.ds(0, 16)] = z16
        return carry

    lax.fori_loop(0, ROWS_PT, _zero, 0)
    r0 = s * ROWS_PT
    pltpu.sync_copy(dbuf, dacc.at[pl.ds(r0, ROWS_PT)])
    plsc.subcore_barrier()

    def _step(t, carry):
        sd = []
        for k in range(K):
            ch = t * K + k
            sd.append(pltpu.async_copy(ones, dacc.at[sdst.at[ch]],
                                       sem, add=True))
        for dd in sd:
            dd.wait()
        return carry

    lax.fori_loop(0, CH // K, _step, 0)
    plsc.subcore_barrier()

    pltpu.sync_copy(dacc.at[pl.ds(r0, ROWS_PT)], dbuf)
    pltpu.sync_copy(dbuf, out_hbm.at[c, pl.ds(r0, ROWS_PT)])


def _l2n(x):
    return x / jnp.maximum(jnp.sqrt(jnp.sum(x * x, axis=1, keepdims=True)),
                           1e-12)


def _tc_embed_body(feat_ref, wm_ref, bm_ref, pref_ref, w1_ref, t1_ref):
    nf = jnp.dot(feat_ref[...], wm_ref[...],
                 preferred_element_type=_f32) + bm_ref[...]
    xu = _l2n(pref_ref[...])
    xl = _l2n(nf)
    tu = jnp.dot(xu, w1_ref[...], preferred_element_type=_f32)
    tl = jnp.dot(xl, w1_ref[...], preferred_element_type=_f32)
    t1_ref[...] = jnp.concatenate([tu, tl], axis=1)


def _tc_dis_body(degp_ref, t1_ref, g1_ref, dis_ref):
    degm = degp_ref[0, 0:N_NODES, :] + degp_ref[1, 0:N_NODES, :]
    deg = jnp.sum(degm, axis=1, keepdims=True) * (1.0 / DEGW)
    dis = jnp.where(deg > 0.0, lax.rsqrt(jnp.maximum(deg, 1e-12)), 0.0)
    dis_ref[...] = dis
    t = t1_ref[...]
    gu = dis[0:N_USER] * t[:, 0:D]
    gl = dis[N_USER:N_NODES] * t[:, D:2 * D]
    g = jnp.concatenate([gu, gl], axis=1)
    g1_ref[...] = jnp.concatenate(
        [g, jnp.zeros((N_PAD // 2 - N_USER, 2 * D), _f32)], axis=0)


def _tc_mid_body(s1_ref, dis_ref, w2_ref, p1_ref, p2_ref, g2_ref):
    sp = s1_ref[0, 0:N_USER, :] + s1_ref[1, 0:N_USER, :]
    dis_u = dis_ref[0:N_USER]
    dis_l = dis_ref[N_USER:N_NODES]
    tu = dis_u * sp[:, 0:D]
    tl = dis_l * sp[:, D:2 * D]
    scale = p1_ref[0, 0] * p2_ref[0, 0]
    gu = (scale * dis_u) * jnp.dot(tu, w2_ref[...],
                                   preferred_element_type=_f32)
    gl = (scale * dis_l) * jnp.dot(tl, w2_ref[...],
                                   preferred_element_type=_f32)
    g = jnp.concatenate([gu, gl], axis=1)
    g2_ref[...] = jnp.concatenate(
        [g, jnp.zeros((N_PAD // 2 - N_USER, 2 * D), _f32)], axis=0)


def _tc_tail_body(s2_ref, dis_ref, x2_ref):
    sp = s2_ref[0, 0:N_USER, :] + s2_ref[1, 0:N_USER, :]
    xu = dis_ref[0:N_USER] * sp[:, 0:D]
    xl = dis_ref[N_USER:N_NODES] * sp[:, D:2 * D]
    x2_ref[...] = jnp.concatenate([xu, xl], axis=0)


def _sc_mesh():
    return plsc.VectorSubcoreMesh(core_axis_name="c", subcore_axis_name="s")


def _sc_scatter(g, src, dst):
    return pl.kernel(
        _sc_scatter_body,
        out_type=jax.ShapeDtypeStruct((NC, N_PAD, D), _f32),
        mesh=_sc_mesh(),
        scratch_types=[
            pltpu.VMEM((CH, B), jnp.int32),
            pltpu.VMEM((CH, B), jnp.int32),
            pltpu.VMEM((2, K, B, D), _f32),
            pltpu.VMEM_SHARED((N_PAD, D), _f32),
            pltpu.SemaphoreType.DMA,
            pltpu.SemaphoreType.DMA,
        ],
        compiler_params=pltpu.CompilerParams(use_tc_tiling_on_sc=False),
    )(g, src, dst)


def _sc_degree(dst):
    return pl.kernel(
        _sc_degree_body,
        out_type=jax.ShapeDtypeStruct((NC, N_PAD, DEGW), _f32),
        mesh=_sc_mesh(),
        scratch_types=[
            pltpu.VMEM((CH, B), jnp.int32),
            pltpu.VMEM((B, DEGW), _f32),
            pltpu.VMEM((ROWS_PT, DEGW), _f32),
            pltpu.VMEM_SHARED((N_PAD, DEGW), _f32),
            pltpu.SemaphoreType.DMA,
        ],
        compiler_params=pltpu.CompilerParams(use_tc_tiling_on_sc=False),
    )(dst)


def kernel(features, edge_index, preference, W_mlp, b_mlp, W1, p1, W2, p2):
    padv = PAD_NODE + (jnp.arange(E_PAD - N_EDGES, dtype=jnp.int32) % 120)
    ei = jnp.concatenate(
        [edge_index, jnp.broadcast_to(padv, (2, E_PAD - N_EDGES))], axis=1)
    pei = jnp.where(ei < N_USER, ei + ei, ei + ei - (2 * N_USER - 1))
    src = pei[0].reshape(NW * CH, B)
    dst = pei[1].reshape(NW * CH, B)
    dst_raw = ei[1].reshape(NW * CH, B)

    deg_parts = _sc_degree(dst_raw)
    t1p = pl.pallas_call(
        _tc_embed_body,
        out_shape=jax.ShapeDtypeStruct((N_USER, 2 * D), _f32),
    )(features, W_mlp, b_mlp.reshape(1, D), preference, W1)
    g1p, dis = pl.pallas_call(
        _tc_dis_body,
        out_shape=[jax.ShapeDtypeStruct((N_PAD // 2, 2 * D), _f32),
                   jax.ShapeDtypeStruct((N_NODES, 1), _f32)],
    )(deg_parts, t1p)

    s1 = _sc_scatter(g1p.reshape(N_PAD, D), src, dst)
    g2p = pl.pallas_call(
        _tc_mid_body,
        out_shape=jax.ShapeDtypeStruct((N_PAD // 2, 2 * D), _f32),
    )(s1.reshape(NC, N_PAD // 2, 2 * D), dis, W2,
      p1.reshape(1, 1), p2.reshape(1, 1))

    s2 = _sc_scatter(g2p.reshape(N_PAD, D), src, dst)
    x2 = pl.pallas_call(
        _tc_tail_body,
        out_shape=jax.ShapeDtypeStruct((N_NODES, D), _f32),
    )(s2.reshape(NC, N_PAD // 2, 2 * D), dis)
    return (x2, p2)


# speedup vs baseline: 41.7820x; 1.0192x over previous
import jax
import jax.numpy as jnp
from jax import lax
from jax.experimental import pallas as pl
from jax.experimental.pallas import tpu as pltpu
from jax.experimental.pallas import tpu_sc as plsc

N_USER = 5000
N_NODES = 10000
N_PAD = 10240
N_EDGES = 320000
D = 64
DEGW = 16

NC, NS = 2, 16
NW = NC * NS
B = 128
E_PAD = 327680
PAD_NODE = 10000
EPW = E_PAD // NW
CH = EPW // B
K = 4
ROWS_PT = N_PAD // NS

_f32 = jnp.float32


def _sc_scatter_body(g_hbm, src_hbm, dst_hbm, out_hbm,
                     ssrc, sdst, rows, acc, gsem, ssem):
    c = lax.axis_index("c")
    s = lax.axis_index("s")
    wid = c * NS + s

    d1 = pltpu.async_copy(src_hbm.at[pl.ds(wid * CH, CH)], ssrc, gsem)
    d2 = pltpu.async_copy(dst_hbm.at[pl.ds(wid * CH, CH)], sdst, ssem)

    z = jnp.zeros((16,), _f32)
    zref = rows.at[0, 0]

    def _zrow(i, carry):
        for j in range(D // 16):
            zref[i, pl.ds(j * 16, 16)] = z
        return carry

    lax.fori_loop(0, B, _zrow, 0)
    r0 = s * ROWS_PT
    zz = []
    for j in range(ROWS_PT // B):
        zz.append(pltpu.async_copy(rows.at[0, 0],
                                   acc.at[pl.ds(r0 + j * B, B)], gsem))
    d1.wait()
    d2.wait()
    for dd in zz:
        dd.wait()
    plsc.subcore_barrier()

    GR = CH // K

    def _drain_gathers(bank):
        for k in range(K):
            pltpu.make_async_copy(g_hbm.at[pl.ds(0, B)],
                                  rows.at[bank, k], gsem).wait()

    def _drain_scatters(bank):
        for k in range(K):
            pltpu.make_async_copy(rows.at[bank, k],
                                  acc.at[pl.ds(0, B)], ssem).wait()

    def _issue_gathers(t, bank):
        for k in range(K):
            pltpu.async_copy(g_hbm.at[ssrc.at[t * K + k]],
                             rows.at[bank, k], gsem)

    def _issue_scatters(t, bank):
        for k in range(K):
            pltpu.async_copy(rows.at[bank, k], acc.at[sdst.at[t * K + k]],
                             ssem, add=True)

    _issue_gathers(0, 0)

    def _step(t, carry):
        b = lax.rem(t, 2)
        b2 = 1 - b

        @pl.when(t >= 1)
        def _():
            _drain_scatters(b2)

        @pl.when(t <= GR - 2)
        def _():
            _issue_gathers(t + 1, b2)

        _drain_gathers(b)
        _issue_scatters(t, b)
        return carry

    lax.fori_loop(0, GR, _step, 0)
    _drain_scatters(lax.rem(GR - 1, 2))
    plsc.subcore_barrier()

    pltpu.sync_copy(acc.at[pl.ds(r0, ROWS_PT)],
                    out_hbm.at[c, pl.ds(r0, ROWS_PT)])


def _sc_degree_body(dst_hbm, out_hbm, sdst, ones, dbuf, dacc, sem):
    c = lax.axis_index("c")
    s = lax.axis_index("s")
    wid = c * NS + s

    pltpu.sync_copy(dst_hbm.at[pl.ds(wid * CH, CH)], sdst)

    one16 = jnp.full((16,), 1.0, _f32)
    z16 = jnp.zeros((16,), _f32)

    def _fill(i, carry):
        ones[i, pl.ds(0, 16)] = one16
        return carry

    lax.fori_loop(0, B, _fill, 0)

    def _zero(i, carry):
        dbuf[i, pl.ds(0, 16)] = z16
        return carry

    lax.fori_loop(0, ROWS_PT, _zero, 0)
    r0 = s * ROWS_PT
    pltpu.sync_copy(dbuf, dacc.at[pl.ds(r0, ROWS_PT)])
    plsc.subcore_barrier()

    def _step(t, carry):
        sd = []
        for k in range(K):
            ch = t * K + k
            sd.append(pltpu.async_copy(ones, dacc.at[sdst.at[ch]],
                                       sem, add=True))
        for dd in sd:
            dd.wait()
        return carry

    lax.fori_loop(0, CH // K, _step, 0)
    plsc.subcore_barrier()

    pltpu.sync_copy(dacc.at[pl.ds(r0, ROWS_PT)],
                    out_hbm.at[c, pl.ds(r0, ROWS_PT)])


def _l2n(x):
    return x / jnp.maximum(jnp.sqrt(jnp.sum(x * x, axis=1, keepdims=True)),
                           1e-12)


def _tc_embed_body(feat_ref, wm_ref, bm_ref, pref_ref, w1_ref, t1_ref):
    nf = jnp.dot(feat_ref[...], wm_ref[...],
                 preferred_element_type=_f32) + bm_ref[...]
    xu = _l2n(pref_ref[...])
    xl = _l2n(nf)
    tu = jnp.dot(xu, w1_ref[...], preferred_element_type=_f32)
    tl = jnp.dot(xl, w1_ref[...], preferred_element_type=_f32)
    t1_ref[...] = jnp.concatenate([tu, tl], axis=1)


def _tc_dis_body(degp_ref, t1_ref, g1_ref, dis_ref):
    degm = degp_ref[0, 0:N_NODES, :] + degp_ref[1, 0:N_NODES, :]
    deg = jnp.sum(degm, axis=1, keepdims=True) * (1.0 / DEGW)
    dis = jnp.where(deg > 0.0, lax.rsqrt(jnp.maximum(deg, 1e-12)), 0.0)
    dis_ref[...] = dis
    t = t1_ref[...]
    gu = dis[0:N_USER] * t[:, 0:D]
    gl = dis[N_USER:N_NODES] * t[:, D:2 * D]
    g = jnp.concatenate([gu, gl], axis=1)
    g1_ref[...] = jnp.concatenate(
        [g, jnp.zeros((N_PAD // 2 - N_USER, 2 * D), _f32)], axis=0)


def _tc_mid_body(s1_ref, dis_ref, w2_ref, p1_ref, p2_ref, g2_ref):
    sp = s1_ref[0, 0:N_USER, :] + s1_ref[1, 0:N_USER, :]
    dis_u = dis_ref[0:N_USER]
    dis_l = dis_ref[N_USER:N_NODES]
    tu = dis_u * sp[:, 0:D]
    tl = dis_l * sp[:, D:2 * D]
    scale = p1_ref[0, 0] * p2_ref[0, 0]
    gu = (scale * dis_u) * jnp.dot(tu, w2_ref[...],
                                   preferred_element_type=_f32)
    gl = (scale * dis_l) * jnp.dot(tl, w2_ref[...],
                                   preferred_element_type=_f32)
    g = jnp.concatenate([gu, gl], axis=1)
    g2_ref[...] = jnp.concatenate(
        [g, jnp.zeros((N_PAD // 2 - N_USER, 2 * D), _f32)], axis=0)


def _tc_tail_body(s2_ref, dis_ref, x2_ref):
    sp = s2_ref[0, 0:N_USER, :] + s2_ref[1, 0:N_USER, :]
    xu = dis_ref[0:N_USER] * sp[:, 0:D]
    xl = dis_ref[N_USER:N_NODES] * sp[:, D:2 * D]
    x2_ref[...] = jnp.concatenate([xu, xl], axis=0)


def _sc_mesh():
    return plsc.VectorSubcoreMesh(core_axis_name="c", subcore_axis_name="s")


def _sc_scatter(g, src, dst):
    return pl.kernel(
        _sc_scatter_body,
        out_type=jax.ShapeDtypeStruct((NC, N_PAD, D), _f32),
        mesh=_sc_mesh(),
        scratch_types=[
            pltpu.VMEM((CH, B), jnp.int32),
            pltpu.VMEM((CH, B), jnp.int32),
            pltpu.VMEM((2, K, B, D), _f32),
            pltpu.VMEM_SHARED((N_PAD, D), _f32),
            pltpu.SemaphoreType.DMA,
            pltpu.SemaphoreType.DMA,
        ],
        compiler_params=pltpu.CompilerParams(use_tc_tiling_on_sc=False),
    )(g, src, dst)


def _sc_degree(dst):
    return pl.kernel(
        _sc_degree_body,
        out_type=jax.ShapeDtypeStruct((NC, N_PAD, DEGW), _f32),
        mesh=_sc_mesh(),
        scratch_types=[
            pltpu.VMEM((CH, B), jnp.int32),
            pltpu.VMEM((B, DEGW), _f32),
            pltpu.VMEM((ROWS_PT, DEGW), _f32),
            pltpu.VMEM_SHARED((N_PAD, DEGW), _f32),
            pltpu.SemaphoreType.DMA,
        ],
        compiler_params=pltpu.CompilerParams(use_tc_tiling_on_sc=False),
    )(dst)


def kernel(features, edge_index, preference, W_mlp, b_mlp, W1, p1, W2, p2):
    padv = PAD_NODE + (jnp.arange(E_PAD - N_EDGES, dtype=jnp.int32) % 120)
    ei = jnp.concatenate(
        [edge_index, jnp.broadcast_to(padv, (2, E_PAD - N_EDGES))], axis=1)
    pei = jnp.where(ei < N_USER, ei + ei, ei + ei - (2 * N_USER - 1))
    src = pei[0].reshape(NW * CH, B)
    dst = pei[1].reshape(NW * CH, B)
    dst_raw = ei[1].reshape(NW * CH, B)

    deg_parts = _sc_degree(dst_raw)
    t1p = pl.pallas_call(
        _tc_embed_body,
        out_shape=jax.ShapeDtypeStruct((N_USER, 2 * D), _f32),
    )(features, W_mlp, b_mlp.reshape(1, D), preference, W1)
    g1p, dis = pl.pallas_call(
        _tc_dis_body,
        out_shape=[jax.ShapeDtypeStruct((N_PAD // 2, 2 * D), _f32),
                   jax.ShapeDtypeStruct((N_NODES, 1), _f32)],
    )(deg_parts, t1p)

    s1 = _sc_scatter(g1p.reshape(N_PAD, D), src, dst)
    g2p = pl.pallas_call(
        _tc_mid_body,
        out_shape=jax.ShapeDtypeStruct((N_PAD // 2, 2 * D), _f32),
    )(s1.reshape(NC, N_PAD // 2, 2 * D), dis, W2,
      p1.reshape(1, 1), p2.reshape(1, 1))

    s2 = _sc_scatter(g2p.reshape(N_PAD, D), src, dst)
    x2 = pl.pallas_call(
        _tc_tail_body,
        out_shape=jax.ShapeDtypeStruct((N_NODES, D), _f32),
    )(s2.reshape(NC, N_PAD // 2, 2 * D), dis)
    return (x2, p2)
